# baseline (speedup 1.0000x reference)
import functools

import jax
import jax.numpy as jnp
from jax import lax
from jax.experimental import pallas as pl
from jax.experimental.pallas import tpu as pltpu
from jax.experimental.pallas import tpu_sc as plsc

_N = 10000
_E = 320000
_D_IN = 128
_D_H = 64
_D_OUT = 128
_D_M = 16
_LAM = 0.25

_NC = 2
_NS = 16
_L = 16
_NW = _NC * _NS
_EPT = _E // _NW
_CE = 80
_NCHUNK = _EPT // _CE
_NPAD = 10240
_RPS = _NPAD // _NS

_mesh = plsc.VectorSubcoreMesh(core_axis_name="c", subcore_axis_name="s")
_sc_params = pltpu.CompilerParams(
    needs_layout_passes=False, use_tc_tiling_on_sc=False
)


@functools.partial(
    pl.kernel,
    out_type=[
        jax.ShapeDtypeStruct((_NW, _NCHUNK, _CE), jnp.float32),
        jax.ShapeDtypeStruct((_NC, _NPAD), jnp.float32),
    ],
    mesh=_mesh,
    compiler_params=_sc_params,
    scratch_types=[
        pltpu.VMEM((_N * _D_M // 2,), jnp.int32),
        pltpu.VMEM((_NCHUNK, _CE), jnp.int32),
        pltpu.VMEM((_NCHUNK, _CE), jnp.int32),
        pltpu.VMEM((_NCHUNK, _CE), jnp.float32),
        pltpu.VMEM((_RPS,), jnp.float32),
        pltpu.VMEM_SHARED((_NPAD,), jnp.float32),
        pltpu.SemaphoreType.DMA,
        pltpu.SemaphoreType.DMA,
    ],
)
def _sc_edge(mnp_hbm, src_hbm, dst_hbm, ew_hbm, deg_hbm,
             mn_v, src_v, dst_v, ew_v, zb_v, deg_sp, sem, sem_deg):
    cid = lax.axis_index("c")
    sid = lax.axis_index("s")
    wid = sid * _NC + cid

    for i in range(_RPS // _L):
        zb_v[pl.ds(i * _L, _L)] = jnp.zeros((_L,), jnp.float32)
    pltpu.sync_copy(zb_v, deg_sp.at[pl.ds(sid * _RPS, _RPS)])

    cp0 = pltpu.async_copy(mnp_hbm, mn_v, sem)
    cp1 = pltpu.async_copy(src_hbm.at[wid], src_v, sem)
    cp2 = pltpu.async_copy(dst_hbm.at[wid], dst_v, sem)
    cp0.wait()
    cp1.wait()
    cp2.wait()
    plsc.subcore_barrier()

    def chunk(j, _):
        for t in range(_CE // _L):
            s16 = src_v[j, pl.ds(t * _L, _L)]
            d16 = dst_v[j, pl.ds(t * _L, _L)]
            fs = s16 * (_D_M // 2)
            fd = d16 * (_D_M // 2)
            acc = None
            for k in range(_D_M // 2):
                u = plsc.load_gather(mn_v, [fs + k])
                v = plsc.load_gather(mn_v, [fd + k])
                p = plsc.bitcast(u, jnp.bfloat16) * plsc.bitcast(v, jnp.bfloat16)
                acc = p if acc is None else acc + p
            alo, ahi = plsc.unpack(acc, format=plsc.PackFormat.INTERLEAVED)
            s = alo + ahi
            sim = jnp.clip((s + 1.0) * 0.5, 0.0, 1.0)
            ew_v[j, pl.ds(t * _L, _L)] = (1.0 - _LAM) + _LAM * sim
        pltpu.async_copy(ew_v.at[j], deg_sp.at[dst_v.at[j]], sem_deg, add=True)
        return 0

    lax.fori_loop(0, _NCHUNK, chunk, 0)
    pltpu.sync_copy(ew_v, ew_hbm.at[wid])
    pltpu.make_async_copy(ew_hbm.at[wid], ew_v, sem_deg).wait()
    plsc.subcore_barrier()
    pltpu.sync_copy(deg_sp.at[pl.ds(sid * _RPS, _RPS)],
                    deg_hbm.at[cid, pl.ds(sid * _RPS, _RPS)])


_NBUF = 5


@functools.partial(
    pl.kernel,
    out_type=jax.ShapeDtypeStruct((_NC, _NPAD, _D_H), jnp.float32),
    mesh=_mesh,
    compiler_params=_sc_params,
    scratch_types=[
        pltpu.VMEM((_NCHUNK, _CE), jnp.int32),
        pltpu.VMEM((_NCHUNK, _CE), jnp.int32),
        pltpu.VMEM((_NCHUNK, _CE), jnp.float32),
        pltpu.VMEM((_NBUF, _CE, _D_H), jnp.float32),
        pltpu.VMEM_SHARED((_NPAD, _D_H), jnp.float32),
        pltpu.SemaphoreType.DMA,
        [pltpu.SemaphoreType.DMA] * _NBUF,
        [pltpu.SemaphoreType.DMA] * _NBUF,
    ],
)
def _sc_mp(g_hbm, src_hbm, dst_hbm, ew_hbm, acc_hbm,
           src_v, dst_v, ew_v, rows_v, acc_sp, sem, semg, sems):
    cid = lax.axis_index("c")
    sid = lax.axis_index("s")
    wid = sid * _NC + cid

    nsl = pl.ds(sid * _RPS, _RPS)
    pltpu.sync_copy(g_hbm.at[nsl], acc_sp.at[nsl])
    cp1 = pltpu.async_copy(src_hbm.at[wid], src_v, sem)
    cp2 = pltpu.async_copy(dst_hbm.at[wid], dst_v, sem)
    cp3 = pltpu.async_copy(ew_hbm.at[wid], ew_v, sem)
    cp1.wait()
    cp2.wait()
    cp3.wait()
    plsc.subcore_barrier()

    def gather(k, b):
        h = _CE // 2
        pltpu.async_copy(g_hbm.at[src_v.at[k, pl.ds(0, h)]],
                         rows_v.at[b, pl.ds(0, h)], semg[b])
        pltpu.async_copy(g_hbm.at[src_v.at[k, pl.ds(h, h)]],
                         rows_v.at[b, pl.ds(h, h)], semg[b])

    def wait_rowb(sem_list, b):
        pltpu.make_async_copy(g_hbm.at[pl.ds(0, _CE)], rows_v.at[b],
                              sem_list[b]).wait()

    def scale_scatter(k, b):
        for t in range(_CE // _L):
            ew16 = ew_v[k, pl.ds(t * _L, _L)]
            for e in range(_L):
                w = ew16[e]
                r = t * _L + e
                for jj in range(_D_H // _L):
                    sl = pl.ds(jj * _L, _L)
                    rows_v[b, r, sl] = rows_v[b, r, sl] * w
        pltpu.async_copy(rows_v.at[b], acc_sp.at[dst_v.at[k]], sems[b],
                         add=True)

    gather(0, 0)
    gather(1, 1)
    gather(2, 2)
    for k0 in range(2):
        gather(k0 + 3, (k0 + 3) % _NBUF)
        wait_rowb(semg, k0)
        scale_scatter(k0, k0)

    def slot(k, b_cur, b_pre):
        wait_rowb(sems, b_pre)
        gather(k + 3, b_pre)
        wait_rowb(semg, b_cur)
        scale_scatter(k, b_cur)

    def body(m, _):
        k = 2 + m * _NBUF
        for bb in range(_NBUF):
            slot(k + bb, (2 + bb) % _NBUF, bb)
        return 0

    lax.fori_loop(0, (_NCHUNK - _NBUF) // _NBUF, body, 0)
    for k0 in range(_NCHUNK - 3, _NCHUNK):
        b_cur = k0 % _NBUF
        wait_rowb(sems, (k0 + 3) % _NBUF)
        wait_rowb(semg, b_cur)
        scale_scatter(k0, b_cur)
    wait_rowb(sems, (_NCHUNK - 2) % _NBUF)
    wait_rowb(sems, (_NCHUNK - 1) % _NBUF)

    plsc.subcore_barrier()
    pltpu.sync_copy(acc_sp.at[pl.ds(sid * _RPS, _RPS)],
                    acc_hbm.at[cid, pl.ds(sid * _RPS, _RPS)])


def _tc_pre_body(x_ref, motif_ref, w1_ref, p1_ref, mn_ref, h1_ref, xp_ref):
    m = motif_ref[...]
    nrm = jnp.sqrt(jnp.sum(m * m, axis=1, keepdims=True))
    mn_ref[...] = m / (nrm + 1e-8)
    xx = x_ref[...]
    h1_ref[...] = jnp.dot(xx, w1_ref[...], preferred_element_type=jnp.float32)
    xp_ref[...] = jnp.dot(xx, p1_ref[...], preferred_element_type=jnp.float32)


_tc_pre = pl.pallas_call(
    _tc_pre_body,
    out_shape=[
        jax.ShapeDtypeStruct((_N, _D_M), jnp.float32),
        jax.ShapeDtypeStruct((_N, _D_H), jnp.float32),
        jax.ShapeDtypeStruct((_N, _D_H), jnp.float32),
    ],
)


def _tc_mid_body(degt_ref, h1_ref, dinv_ref, g1_ref):
    deg = jnp.sum(degt_ref[...], axis=1, keepdims=True) + 1.0
    dinv = lax.rsqrt(deg)
    dinv_ref[...] = dinv
    g1_ref[pl.ds(0, _N), :] = h1_ref[...] * dinv
    g1_ref[pl.ds(_N, _NPAD - _N), :] = jnp.zeros(
        (_NPAD - _N, _D_H), jnp.float32
    )


_tc_mid = pl.pallas_call(
    _tc_mid_body,
    out_shape=[
        jax.ShapeDtypeStruct((_N, 1), jnp.float32),
        jax.ShapeDtypeStruct((_NPAD, _D_H), jnp.float32),
    ],
)


def _ln_relu(h, gam, bet):
    mu = jnp.mean(h, axis=-1, keepdims=True)
    var = jnp.mean((h - mu) ** 2, axis=-1, keepdims=True)
    ln = (h - mu) * lax.rsqrt(var + 1e-5) * gam + bet
    return jnp.maximum(ln, 0.0)


def _tc_l1_body(p_ref, g1v_ref, dinv_ref, xp_ref, b1_ref, gam_ref, bet_ref,
                w2_ref, x1_ref, g2v_ref):
    dinv = dinv_ref[...]
    nsl = pl.ds(0, _N)
    acc = p_ref[0, nsl, :] + p_ref[1, nsl, :] - g1v_ref[nsl, :]
    conv = acc * dinv + b1_ref[...]
    x1 = _ln_relu(conv, gam_ref[...], bet_ref[...]) + xp_ref[...]
    x1_ref[...] = x1
    g2v_ref[nsl, :] = jnp.dot(x1, w2_ref[...],
                              preferred_element_type=jnp.float32) * dinv
    g2v_ref[pl.ds(_N, _NPAD - _N), :] = jnp.zeros(
        (_NPAD - _N, _D_H), jnp.float32
    )


_tc_l1 = pl.pallas_call(
    _tc_l1_body,
    out_shape=[
        jax.ShapeDtypeStruct((_N, _D_H), jnp.float32),
        jax.ShapeDtypeStruct((_NPAD, _D_H), jnp.float32),
    ],
)


def _tc_l2_body(p_ref, g2v_ref, dinv_ref, x1_ref, b2_ref, gam_ref, bet_ref,
                wh_ref, bh_ref, out_ref):
    nsl = pl.ds(0, _N)
    acc = p_ref[0, nsl, :] + p_ref[1, nsl, :] - g2v_ref[nsl, :]
    conv = acc * dinv_ref[...] + b2_ref[...]
    x2 = _ln_relu(conv, gam_ref[...], bet_ref[...]) + x1_ref[...]
    out_ref[...] = jnp.dot(x2, wh_ref[...],
                           preferred_element_type=jnp.float32) + bh_ref[...]


_tc_l2 = pl.pallas_call(
    _tc_l2_body,
    out_shape=jax.ShapeDtypeStruct((_N, _D_OUT), jnp.float32),
)


def kernel(x, edge_index, motif_x, W1, b1, W2, b2, g1, be1, g2, be2, P1, Wh, bh):
    src3 = edge_index[0].reshape(_NW, _NCHUNK, _CE)
    dst3 = edge_index[1].reshape(_NW, _NCHUNK, _CE)
    mn, h1, xP = _tc_pre(x, motif_x, W1, P1)
    mn_pk = lax.bitcast_convert_type(
        mn.astype(jnp.bfloat16).reshape(_N, _D_M // 2, 2), jnp.int32
    ).reshape(_N * _D_M // 2)
    ew3, degp = _sc_edge(mn_pk, src3, dst3)
    dinv, g1v = _tc_mid(degp[:, :_N].T, h1)
    acc1 = _sc_mp(g1v, src3, dst3, ew3)
    x1, g2v = _tc_l1(acc1, g1v, dinv, xP, b1, g1, be1, W2)
    acc2 = _sc_mp(g2v, src3, dst3, ew3)
    return _tc_l2(acc2, g2v, dinv, x1, b2, g2, be2, Wh, bh)


# speedup vs baseline: 1.0292x; 1.0292x over previous
import functools

import jax
import jax.numpy as jnp
from jax import lax
from jax.experimental import pallas as pl
from jax.experimental.pallas import tpu as pltpu
from jax.experimental.pallas import tpu_sc as plsc

_N = 10000
_E = 320000
_D_IN = 128
_D_H = 64
_D_OUT = 128
_D_M = 16
_LAM = 0.25

_NC = 2
_NS = 16
_L = 16
_NW = _NC * _NS
_EPT = _E // _NW
_CE = 80
_NCHUNK = _EPT // _CE
_NPAD = 10240
_RPS = _NPAD // _NS

_mesh = plsc.VectorSubcoreMesh(core_axis_name="c", subcore_axis_name="s")
_sc_params = pltpu.CompilerParams(
    needs_layout_passes=False, use_tc_tiling_on_sc=False
)


@functools.partial(
    pl.kernel,
    out_type=[
        jax.ShapeDtypeStruct((_NW, _NCHUNK, _CE), jnp.float32),
        jax.ShapeDtypeStruct((_NC, _NPAD), jnp.float32),
    ],
    mesh=_mesh,
    compiler_params=_sc_params,
    scratch_types=[
        pltpu.VMEM((_N * _D_M // 2,), jnp.int32),
        pltpu.VMEM((_NCHUNK, _CE), jnp.int32),
        pltpu.VMEM((_NCHUNK, _CE), jnp.int32),
        pltpu.VMEM((_NCHUNK, _CE), jnp.float32),
        pltpu.VMEM((_RPS,), jnp.float32),
        pltpu.VMEM_SHARED((_NPAD,), jnp.float32),
        pltpu.SemaphoreType.DMA,
        pltpu.SemaphoreType.DMA,
    ],
)
def _sc_edge(mnp_hbm, src_hbm, dst_hbm, ew_hbm, deg_hbm,
             mn_v, src_v, dst_v, ew_v, zb_v, deg_sp, sem, sem_deg):
    cid = lax.axis_index("c")
    sid = lax.axis_index("s")
    wid = sid * _NC + cid

    for i in range(_RPS // _L):
        zb_v[pl.ds(i * _L, _L)] = jnp.zeros((_L,), jnp.float32)
    pltpu.sync_copy(zb_v, deg_sp.at[pl.ds(sid * _RPS, _RPS)])

    cp0 = pltpu.async_copy(mnp_hbm, mn_v, sem)
    cp1 = pltpu.async_copy(src_hbm.at[wid], src_v, sem)
    cp2 = pltpu.async_copy(dst_hbm.at[wid], dst_v, sem)
    cp0.wait()
    cp1.wait()
    cp2.wait()
    plsc.subcore_barrier()

    def chunk(j, _):
        for t in range(_CE // _L):
            s16 = src_v[j, pl.ds(t * _L, _L)]
            d16 = dst_v[j, pl.ds(t * _L, _L)]
            fs = s16 * (_D_M // 2)
            fd = d16 * (_D_M // 2)
            acc = None
            for k in range(_D_M // 2):
                u = plsc.load_gather(mn_v, [fs + k])
                v = plsc.load_gather(mn_v, [fd + k])
                p = plsc.bitcast(u, jnp.bfloat16) * plsc.bitcast(v, jnp.bfloat16)
                acc = p if acc is None else acc + p
            alo, ahi = plsc.unpack(acc, format=plsc.PackFormat.INTERLEAVED)
            s = alo + ahi
            sim = jnp.clip((s + 1.0) * 0.5, 0.0, 1.0)
            ew_v[j, pl.ds(t * _L, _L)] = (1.0 - _LAM) + _LAM * sim
        pltpu.async_copy(ew_v.at[j], deg_sp.at[dst_v.at[j]], sem_deg, add=True)
        return 0

    lax.fori_loop(0, _NCHUNK, chunk, 0)
    pltpu.sync_copy(ew_v, ew_hbm.at[wid])
    pltpu.make_async_copy(ew_hbm.at[wid], ew_v, sem_deg).wait()
    plsc.subcore_barrier()
    pltpu.sync_copy(deg_sp.at[pl.ds(sid * _RPS, _RPS)],
                    deg_hbm.at[cid, pl.ds(sid * _RPS, _RPS)])


_NBUF = 5


@functools.partial(
    pl.kernel,
    out_type=jax.ShapeDtypeStruct((_NC, _NPAD, _D_H), jnp.float32),
    mesh=_mesh,
    compiler_params=_sc_params,
    scratch_types=[
        pltpu.VMEM((_NCHUNK, _CE), jnp.int32),
        pltpu.VMEM((_NCHUNK, _CE), jnp.int32),
        pltpu.VMEM((_NCHUNK, _CE), jnp.float32),
        pltpu.VMEM((_NBUF, _CE, _D_H), jnp.float32),
        pltpu.VMEM_SHARED((_NPAD, _D_H), jnp.float32),
        pltpu.SemaphoreType.DMA,
        [pltpu.SemaphoreType.DMA] * _NBUF,
        [pltpu.SemaphoreType.DMA] * _NBUF,
    ],
)
def _sc_mp(g_hbm, src_hbm, dst_hbm, ew_hbm, acc_hbm,
           src_v, dst_v, ew_v, rows_v, acc_sp, sem, semg, sems):
    cid = lax.axis_index("c")
    sid = lax.axis_index("s")
    wid = sid * _NC + cid

    nsl = pl.ds(sid * _RPS, _RPS)
    pltpu.sync_copy(g_hbm.at[nsl], acc_sp.at[nsl])
    cp1 = pltpu.async_copy(src_hbm.at[wid], src_v, sem)
    cp2 = pltpu.async_copy(dst_hbm.at[wid], dst_v, sem)
    cp3 = pltpu.async_copy(ew_hbm.at[wid], ew_v, sem)
    cp1.wait()
    cp2.wait()
    cp3.wait()
    plsc.subcore_barrier()

    def gather(k, b):
        h = _CE // 2
        pltpu.async_copy(g_hbm.at[src_v.at[k, pl.ds(0, h)]],
                         rows_v.at[b, pl.ds(0, h)], semg[b])
        pltpu.async_copy(g_hbm.at[src_v.at[k, pl.ds(h, h)]],
                         rows_v.at[b, pl.ds(h, h)], semg[b])

    def wait_rowb(sem_list, b):
        pltpu.make_async_copy(g_hbm.at[pl.ds(0, _CE)], rows_v.at[b],
                              sem_list[b]).wait()

    def scale_scatter(k, b):
        @plsc.parallel_loop(0, _CE // _L, unroll=_CE // _L)
        def _(t):
            ew16 = ew_v[k, pl.ds(t * _L, _L)]
            for e in range(_L):
                w = ew16[e]
                for jj in range(_D_H // _L):
                    sl = pl.ds(jj * _L, _L)
                    rows_v[b, t * _L + e, sl] = rows_v[b, t * _L + e, sl] * w
        pltpu.async_copy(rows_v.at[b], acc_sp.at[dst_v.at[k]], sems[b],
                         add=True)

    gather(0, 0)
    gather(1, 1)
    gather(2, 2)
    for k0 in range(2):
        gather(k0 + 3, (k0 + 3) % _NBUF)
        wait_rowb(semg, k0)
        scale_scatter(k0, k0)

    def slot(k, b_cur, b_pre):
        wait_rowb(sems, b_pre)
        gather(k + 3, b_pre)
        wait_rowb(semg, b_cur)
        scale_scatter(k, b_cur)

    def body(m, _):
        k = 2 + m * _NBUF
        for bb in range(_NBUF):
            slot(k + bb, (2 + bb) % _NBUF, bb)
        return 0

    lax.fori_loop(0, (_NCHUNK - _NBUF) // _NBUF, body, 0)
    for k0 in range(_NCHUNK - 3, _NCHUNK):
        b_cur = k0 % _NBUF
        wait_rowb(sems, (k0 + 3) % _NBUF)
        wait_rowb(semg, b_cur)
        scale_scatter(k0, b_cur)
    wait_rowb(sems, (_NCHUNK - 2) % _NBUF)
    wait_rowb(sems, (_NCHUNK - 1) % _NBUF)

    plsc.subcore_barrier()
    pltpu.sync_copy(acc_sp.at[pl.ds(sid * _RPS, _RPS)],
                    acc_hbm.at[cid, pl.ds(sid * _RPS, _RPS)])


def _tc_pre_body(x_ref, motif_ref, w1_ref, p1_ref, mn_ref, h1_ref, xp_ref):
    m = motif_ref[...]
    nrm = jnp.sqrt(jnp.sum(m * m, axis=1, keepdims=True))
    mn_ref[...] = m / (nrm + 1e-8)
    xx = x_ref[...]
    h1_ref[...] = jnp.dot(xx, w1_ref[...], preferred_element_type=jnp.float32)
    xp_ref[...] = jnp.dot(xx, p1_ref[...], preferred_element_type=jnp.float32)


_tc_pre = pl.pallas_call(
    _tc_pre_body,
    out_shape=[
        jax.ShapeDtypeStruct((_N, _D_M), jnp.float32),
        jax.ShapeDtypeStruct((_N, _D_H), jnp.float32),
        jax.ShapeDtypeStruct((_N, _D_H), jnp.float32),
    ],
)


def _tc_mid_body(degt_ref, h1_ref, dinv_ref, g1_ref):
    deg = jnp.sum(degt_ref[...], axis=1, keepdims=True) + 1.0
    dinv = lax.rsqrt(deg)
    dinv_ref[...] = dinv
    g1_ref[pl.ds(0, _N), :] = h1_ref[...] * dinv
    g1_ref[pl.ds(_N, _NPAD - _N), :] = jnp.zeros(
        (_NPAD - _N, _D_H), jnp.float32
    )


_tc_mid = pl.pallas_call(
    _tc_mid_body,
    out_shape=[
        jax.ShapeDtypeStruct((_N, 1), jnp.float32),
        jax.ShapeDtypeStruct((_NPAD, _D_H), jnp.float32),
    ],
)


def _ln_relu(h, gam, bet):
    mu = jnp.mean(h, axis=-1, keepdims=True)
    var = jnp.mean((h - mu) ** 2, axis=-1, keepdims=True)
    ln = (h - mu) * lax.rsqrt(var + 1e-5) * gam + bet
    return jnp.maximum(ln, 0.0)


def _tc_l1_body(p_ref, g1v_ref, dinv_ref, xp_ref, b1_ref, gam_ref, bet_ref,
                w2_ref, x1_ref, g2v_ref):
    dinv = dinv_ref[...]
    nsl = pl.ds(0, _N)
    acc = p_ref[0, nsl, :] + p_ref[1, nsl, :] - g1v_ref[nsl, :]
    conv = acc * dinv + b1_ref[...]
    x1 = _ln_relu(conv, gam_ref[...], bet_ref[...]) + xp_ref[...]
    x1_ref[...] = x1
    g2v_ref[nsl, :] = jnp.dot(x1, w2_ref[...],
                              preferred_element_type=jnp.float32) * dinv
    g2v_ref[pl.ds(_N, _NPAD - _N), :] = jnp.zeros(
        (_NPAD - _N, _D_H), jnp.float32
    )


_tc_l1 = pl.pallas_call(
    _tc_l1_body,
    out_shape=[
        jax.ShapeDtypeStruct((_N, _D_H), jnp.float32),
        jax.ShapeDtypeStruct((_NPAD, _D_H), jnp.float32),
    ],
)


def _tc_l2_body(p_ref, g2v_ref, dinv_ref, x1_ref, b2_ref, gam_ref, bet_ref,
                wh_ref, bh_ref, out_ref):
    nsl = pl.ds(0, _N)
    acc = p_ref[0, nsl, :] + p_ref[1, nsl, :] - g2v_ref[nsl, :]
    conv = acc * dinv_ref[...] + b2_ref[...]
    x2 = _ln_relu(conv, gam_ref[...], bet_ref[...]) + x1_ref[...]
    out_ref[...] = jnp.dot(x2, wh_ref[...],
                           preferred_element_type=jnp.float32) + bh_ref[...]


_tc_l2 = pl.pallas_call(
    _tc_l2_body,
    out_shape=jax.ShapeDtypeStruct((_N, _D_OUT), jnp.float32),
)


def kernel(x, edge_index, motif_x, W1, b1, W2, b2, g1, be1, g2, be2, P1, Wh, bh):
    src3 = edge_index[0].reshape(_NW, _NCHUNK, _CE)
    dst3 = edge_index[1].reshape(_NW, _NCHUNK, _CE)
    mn, h1, xP = _tc_pre(x, motif_x, W1, P1)
    mn_pk = lax.bitcast_convert_type(
        mn.astype(jnp.bfloat16).reshape(_N, _D_M // 2, 2), jnp.int32
    ).reshape(_N * _D_M // 2)
    ew3, degp = _sc_edge(mn_pk, src3, dst3)
    dinv, g1v = _tc_mid(degp[:, :_N].T, h1)
    acc1 = _sc_mp(g1v, src3, dst3, ew3)
    x1, g2v = _tc_l1(acc1, g1v, dinv, xP, b1, g1, be1, W2)
    acc2 = _sc_mp(g2v, src3, dst3, ew3)
    return _tc_l2(acc2, g2v, dinv, x1, b2, g2, be2, Wh, bh)
